# Initial kernel scaffold; baseline (speedup 1.0000x reference)
import jax
import jax.numpy as jnp
from jax.experimental import pallas as pl


def kernel(z, codebook, q):
    raise NotImplementedError("write your pallas kernel here")


# speedup vs baseline: 1.0364x; 1.0364x over previous
import functools

import jax
import jax.numpy as jnp
from jax import lax
from jax.experimental import pallas as pl
from jax.experimental.pallas import tpu as pltpu
from jax.experimental.pallas import tpu_sc as plsc

V = 8192
C = 64
B = 32
T = 1024
QD = 256
VT = 512
NV = V // VT

NC = 2
NS = 16
NW = NC * NS
GPW = (B * QD) // NW
HPW = (B * T) // NW
RPW = V // NS


def _nn_body(z_ref, cb_ref, idxo_ref, idxd_ref, zn_ref, rn_ref, mo_ref, md_ref,
             sel_ref):
    b = pl.program_id(0)
    v = pl.program_id(1)

    @pl.when((v == 0) & (b == 0))
    def _sel():
        t2 = lax.broadcasted_iota(jnp.int32, (T, T), 0)
        col = lax.broadcasted_iota(jnp.int32, (T, T), 1)
        j = col // QD
        k = col - j * QD
        sel_ref[...] = jnp.where(t2 == 4 * k + j, 1.0, 0.0)

    @pl.when(v == 0)
    def _init():
        zb = z_ref[0]
        cn = jnp.sqrt(jnp.sum(zb * zb, axis=0, keepdims=True))
        zn_ref[...] = zb / jnp.maximum(cn, 1e-12)
        y = jnp.dot(zb, sel_ref[...], precision=lax.Precision.HIGHEST,
                    preferred_element_type=jnp.float32)
        r = (((y[:, :QD] + y[:, QD:2 * QD]) + y[:, 2 * QD:3 * QD])
             + y[:, 3 * QD:]) * 0.25
        rcn = jnp.sqrt(jnp.sum(r * r, axis=0, keepdims=True))
        rn_ref[...] = r / jnp.maximum(rcn, 1e-12)
        mo_ref[...] = jnp.full((1, T), -jnp.inf, jnp.float32)
        md_ref[...] = jnp.full((1, QD), -jnp.inf, jnp.float32)
        idxo_ref[...] = jnp.zeros((1, 1, T), jnp.int32)
        idxd_ref[...] = jnp.zeros((1, 1, QD), jnp.int32)

    cb = cb_ref[...]
    n = jnp.sqrt(jnp.sum(cb * cb, axis=1, keepdims=True))
    cbn = cb / jnp.maximum(n, 1e-12)
    base = v * VT

    simo = jnp.dot(cbn, zn_ref[...], preferred_element_type=jnp.float32)
    mo_t = jnp.max(simo, axis=0, keepdims=True)
    io_t = jnp.min(
        jnp.where(simo == mo_t,
                  lax.broadcasted_iota(jnp.int32, simo.shape, 0) + base, V),
        axis=0, keepdims=True)
    bo = mo_t > mo_ref[...]
    idxo_ref[0] = jnp.where(bo, io_t, idxo_ref[0])
    mo_ref[...] = jnp.where(bo, mo_t, mo_ref[...])

    simd = jnp.dot(cbn, rn_ref[...], preferred_element_type=jnp.float32)
    md_t = jnp.max(simd, axis=0, keepdims=True)
    id_t = jnp.min(
        jnp.where(simd == md_t,
                  lax.broadcasted_iota(jnp.int32, simd.shape, 0) + base, V),
        axis=0, keepdims=True)
    bd = md_t > md_ref[...]
    idxd_ref[0] = jnp.where(bd, id_t, idxd_ref[0])
    md_ref[...] = jnp.where(bd, md_t, md_ref[...])


def _nn_call(z, codebook, interpret=False):
    return pl.pallas_call(
        _nn_body,
        grid=(B, NV),
        in_specs=[
            pl.BlockSpec((1, C, T), lambda b, v: (b, 0, 0)),
            pl.BlockSpec((VT, C), lambda b, v: (v, 0)),
        ],
        out_specs=[
            pl.BlockSpec((1, 1, T), lambda b, v: (b, 0, 0)),
            pl.BlockSpec((1, 1, QD), lambda b, v: (b, 0, 0)),
        ],
        out_shape=[
            jax.ShapeDtypeStruct((B, 1, T), jnp.int32),
            jax.ShapeDtypeStruct((B, 1, QD), jnp.int32),
        ],
        scratch_shapes=[
            pltpu.VMEM((C, T), jnp.float32),
            pltpu.VMEM((C, QD), jnp.float32),
            pltpu.VMEM((1, T), jnp.float32),
            pltpu.VMEM((1, QD), jnp.float32),
            pltpu.VMEM((T, T), jnp.float32),
        ],
        interpret=interpret,
    )(z, codebook)


def _sc_body(idxd_hbm, idxo_hbm, cb_hbm, zq_hbm, counts_hbm,
             idxd_v, rows_v, idx2_v, acc_v, ones_v, shared, sem):
    c = lax.axis_index("c")
    s = lax.axis_index("s")
    wid = s * NC + c

    pltpu.sync_copy(idxd_hbm.at[pl.ds(wid * GPW, GPW)], idxd_v)

    def pb(i, _):
        sl = pl.ds(i * 16, 16)
        idxd_v[sl] = lax.shift_right_logical(idxd_v[sl], 1)
        return 0
    lax.fori_loop(0, GPW // 16, pb, 0)
    pltpu.async_copy(cb_hbm.at[idxd_v], rows_v, sem).wait()
    pltpu.sync_copy(rows_v, zq_hbm.at[pl.ds(wid * GPW, GPW)])

    @pl.when(s == 0)
    def _zero():
        def zb(i, _):
            acc_v[pl.ds(i * 16, 16)] = jnp.zeros((16,), jnp.float32)
            return 0
        lax.fori_loop(0, V // 16, zb, 0)
        pltpu.sync_copy(acc_v, shared)

    def ob(i, _):
        ones_v[pl.ds(i * 16, 16)] = jnp.ones((16,), jnp.float32)
        return 0
    lax.fori_loop(0, HPW // 16, ob, 0)
    pltpu.sync_copy(idxo_hbm.at[wid], idx2_v)
    plsc.subcore_barrier()
    for j in range(HPW // 128):
        pltpu.sync_copy(ones_v.at[pl.ds(j * 128, 128)],
                        shared.at[idx2_v.at[j]], add=True)
    plsc.subcore_barrier()

    @pl.when(s == 0)
    def _drain():
        pltpu.sync_copy(shared, acc_v)
        pltpu.sync_copy(acc_v, counts_hbm.at[c])


def _sc_call(idx_ds, idx_org, codebook):
    mesh = plsc.VectorSubcoreMesh(core_axis_name="c", subcore_axis_name="s")
    f = functools.partial(
        pl.kernel, mesh=mesh,
        out_type=[
            jax.ShapeDtypeStruct((B * QD, 2 * C), jnp.float32),
            jax.ShapeDtypeStruct((NC, V), jnp.float32),
        ],
        scratch_types=[
            pltpu.VMEM((GPW,), jnp.int32),
            pltpu.VMEM((GPW, 2 * C), jnp.float32),
            pltpu.VMEM((HPW // 128, 128), jnp.int32),
            pltpu.VMEM((V,), jnp.float32),
            pltpu.VMEM((HPW,), jnp.float32),
            pltpu.VMEM_SHARED((V,), jnp.float32),
            pltpu.SemaphoreType.DMA,
        ],
    )(_sc_body)
    return f(idx_ds, idx_org.reshape(NW, HPW // 128, 128),
             codebook.reshape(V // 2, 2 * C))


def _interp_body(zq_ref, par_ref, counts_ref, zhat_ref, perp_ref, w_ref):
    b = pl.program_id(0)

    @pl.when(b == 0)
    def _init():
        t2 = lax.broadcasted_iota(jnp.int32, (QD, T), 1).astype(jnp.float32)
        k2 = lax.broadcasted_iota(jnp.int32, (QD, T), 0)
        pos = jnp.clip((t2 + 0.5) * (QD / T) - 0.5, 0.0, QD - 1.0)
        lo = jnp.floor(pos)
        loi = lo.astype(jnp.int32)
        hii = jnp.minimum(loi + 1, QD - 1)
        w = pos - lo
        w_ref[...] = (jnp.where(k2 == loi, 1.0 - w, 0.0)
                      + jnp.where(k2 == hii, w, 0.0))
        cts = counts_ref[0:1, :] + counts_ref[1:2, :]
        prob = cts / jnp.sum(cts)
        ent = jnp.sum(prob * jnp.log(prob + 1e-07))
        perp_ref[...] = jnp.exp(-ent).reshape(1, 1)

    rows = zq_ref[0]
    zq = jnp.where(par_ref[0] == 0, rows[:, :C], rows[:, C:])
    zhat_ref[0] = lax.dot_general(
        zq, w_ref[...], (((0,), (0,)), ((), ())),
        precision=lax.Precision.HIGHEST,
        preferred_element_type=jnp.float32)


def _interp_call(zq, par, counts, interpret=False):
    return pl.pallas_call(
        _interp_body,
        grid=(B,),
        in_specs=[
            pl.BlockSpec((1, QD, 2 * C), lambda b: (b, 0, 0)),
            pl.BlockSpec((1, QD, 1), lambda b: (b, 0, 0)),
            pl.BlockSpec((NC, V), lambda b: (0, 0)),
        ],
        out_specs=[
            pl.BlockSpec((1, C, T), lambda b: (b, 0, 0)),
            pl.BlockSpec((1, 1), lambda b: (0, 0)),
        ],
        out_shape=[
            jax.ShapeDtypeStruct((B, C, T), jnp.float32),
            jax.ShapeDtypeStruct((1, 1), jnp.float32),
        ],
        scratch_shapes=[pltpu.VMEM((QD, T), jnp.float32)],
        interpret=interpret,
    )(zq, par, counts)


def kernel(z, codebook, q):
    idx_org3, idx_ds3 = _nn_call(z, codebook)
    idx_org = idx_org3.reshape(-1)
    idx_ds = idx_ds3.reshape(-1) + (q - QD)
    zq, counts = _sc_call(idx_ds.astype(jnp.int32), idx_org, codebook)
    par = (idx_ds & 1).reshape(B, QD, 1)
    z_hat, perp = _interp_call(zq.reshape(B, QD, 2 * C), par, counts)
    return z_hat, perp.reshape(())


# speedup vs baseline: 1.0530x; 1.0161x over previous
import functools

import jax
import jax.numpy as jnp
from jax import lax
from jax.experimental import pallas as pl
from jax.experimental.pallas import tpu as pltpu
from jax.experimental.pallas import tpu_sc as plsc

V = 8192
C = 64
B = 32
T = 1024
QD = 256
VT = 512
NV = V // VT

NC = 2
NS = 16
NW = NC * NS
GPW = (B * QD) // NW
HPW = (B * T) // NW
RPW = V // NS


def _cbn_body(cb_ref, cbn_ref):
    cb = cb_ref[...]
    n = jnp.sqrt(jnp.sum(cb * cb, axis=1, keepdims=True))
    cbn_ref[...] = (cb / jnp.maximum(n, 1e-12)).astype(jnp.bfloat16)


def _cbn_call(codebook, interpret=False):
    return pl.pallas_call(
        _cbn_body,
        grid=(NV,),
        in_specs=[pl.BlockSpec((VT, C), lambda v: (v, 0))],
        out_specs=pl.BlockSpec((VT, C), lambda v: (v, 0)),
        out_shape=jax.ShapeDtypeStruct((V, C), jnp.bfloat16),
        interpret=interpret,
    )(codebook)


def _nn_body(z_ref, cb_ref, idxo_ref, idxd_ref, zn_ref, rn_ref, mo_ref, md_ref,
             sel_ref, iota_ref):
    b = pl.program_id(0)
    v = pl.program_id(1)

    @pl.when((v == 0) & (b == 0))
    def _sel():
        t2 = lax.broadcasted_iota(jnp.int32, (T, T), 0)
        col = lax.broadcasted_iota(jnp.int32, (T, T), 1)
        j = col // QD
        k = col - j * QD
        sel_ref[...] = jnp.where(t2 == 4 * k + j, 1.0, 0.0)
        iota_ref[...] = lax.broadcasted_iota(jnp.int32, (VT, T), 0)

    @pl.when(v == 0)
    def _init():
        zb = z_ref[0]
        cn = jnp.sqrt(jnp.sum(zb * zb, axis=0, keepdims=True))
        zn_ref[...] = (zb / jnp.maximum(cn, 1e-12)).astype(jnp.bfloat16)
        y = jnp.dot(zb, sel_ref[...], precision=lax.Precision.HIGHEST,
                    preferred_element_type=jnp.float32)
        r = (((y[:, :QD] + y[:, QD:2 * QD]) + y[:, 2 * QD:3 * QD])
             + y[:, 3 * QD:]) * 0.25
        rcn = jnp.sqrt(jnp.sum(r * r, axis=0, keepdims=True))
        rn_ref[...] = (r / jnp.maximum(rcn, 1e-12)).astype(jnp.bfloat16)
        mo_ref[...] = jnp.full((1, T), -jnp.inf, jnp.float32)
        md_ref[...] = jnp.full((1, QD), -jnp.inf, jnp.float32)
        idxo_ref[...] = jnp.zeros((1, 1, T), jnp.int32)
        idxd_ref[...] = jnp.zeros((1, 1, QD), jnp.int32)

    cbn = cb_ref[...]
    base = v * VT
    big = jnp.int32(V)

    simo = jnp.dot(cbn, zn_ref[...], preferred_element_type=jnp.float32)
    mo_t = jnp.max(simo, axis=0, keepdims=True)
    io_t = jnp.min(
        jnp.where(simo == mo_t, iota_ref[...], big), axis=0, keepdims=True)
    bo = mo_t > mo_ref[...]
    idxo_ref[0] = jnp.where(bo, io_t + base, idxo_ref[0])
    mo_ref[...] = jnp.where(bo, mo_t, mo_ref[...])

    simd = jnp.dot(cbn, rn_ref[...], preferred_element_type=jnp.float32)
    md_t = jnp.max(simd, axis=0, keepdims=True)
    id_t = jnp.min(
        jnp.where(simd == md_t, iota_ref[:, :QD], big), axis=0, keepdims=True)
    bd = md_t > md_ref[...]
    idxd_ref[0] = jnp.where(bd, id_t + base, idxd_ref[0])
    md_ref[...] = jnp.where(bd, md_t, md_ref[...])


def _nn_call(z, cbn, interpret=False):
    return pl.pallas_call(
        _nn_body,
        grid=(B, NV),
        in_specs=[
            pl.BlockSpec((1, C, T), lambda b, v: (b, 0, 0)),
            pl.BlockSpec((VT, C), lambda b, v: (v, 0)),
        ],
        out_specs=[
            pl.BlockSpec((1, 1, T), lambda b, v: (b, 0, 0)),
            pl.BlockSpec((1, 1, QD), lambda b, v: (b, 0, 0)),
        ],
        out_shape=[
            jax.ShapeDtypeStruct((B, 1, T), jnp.int32),
            jax.ShapeDtypeStruct((B, 1, QD), jnp.int32),
        ],
        scratch_shapes=[
            pltpu.VMEM((C, T), jnp.bfloat16),
            pltpu.VMEM((C, QD), jnp.bfloat16),
            pltpu.VMEM((1, T), jnp.float32),
            pltpu.VMEM((1, QD), jnp.float32),
            pltpu.VMEM((T, T), jnp.float32),
            pltpu.VMEM((VT, T), jnp.int32),
        ],
        interpret=interpret,
    )(z, cbn)


def _sc_body(idxd_hbm, idxo_hbm, cb_hbm, zq_hbm, counts_hbm,
             idxd_v, rows_v, idx2_v, acc_v, ones_v, shared, sem):
    c = lax.axis_index("c")
    s = lax.axis_index("s")
    wid = s * NC + c

    pltpu.sync_copy(idxd_hbm.at[pl.ds(wid * GPW, GPW)], idxd_v)

    def pb(i, _):
        sl = pl.ds(i * 16, 16)
        idxd_v[sl] = lax.shift_right_logical(idxd_v[sl], 1)
        return 0
    lax.fori_loop(0, GPW // 16, pb, 0)
    pltpu.async_copy(cb_hbm.at[idxd_v], rows_v, sem).wait()
    pltpu.sync_copy(rows_v, zq_hbm.at[pl.ds(wid * GPW, GPW)])

    @pl.when(s == 0)
    def _zero():
        def zb(i, _):
            acc_v[pl.ds(i * 16, 16)] = jnp.zeros((16,), jnp.float32)
            return 0
        lax.fori_loop(0, V // 16, zb, 0)
        pltpu.sync_copy(acc_v, shared)

    def ob(i, _):
        ones_v[pl.ds(i * 16, 16)] = jnp.ones((16,), jnp.float32)
        return 0
    lax.fori_loop(0, HPW // 16, ob, 0)
    pltpu.sync_copy(idxo_hbm.at[wid], idx2_v)
    plsc.subcore_barrier()
    for j in range(HPW // 128):
        pltpu.sync_copy(ones_v.at[pl.ds(j * 128, 128)],
                        shared.at[idx2_v.at[j]], add=True)
    plsc.subcore_barrier()

    @pl.when(s == 0)
    def _drain():
        pltpu.sync_copy(shared, acc_v)
        pltpu.sync_copy(acc_v, counts_hbm.at[c])


def _sc_call(idx_ds, idx_org, codebook):
    mesh = plsc.VectorSubcoreMesh(core_axis_name="c", subcore_axis_name="s")
    f = functools.partial(
        pl.kernel, mesh=mesh,
        out_type=[
            jax.ShapeDtypeStruct((B * QD, 2 * C), jnp.float32),
            jax.ShapeDtypeStruct((NC, V), jnp.float32),
        ],
        scratch_types=[
            pltpu.VMEM((GPW,), jnp.int32),
            pltpu.VMEM((GPW, 2 * C), jnp.float32),
            pltpu.VMEM((HPW // 128, 128), jnp.int32),
            pltpu.VMEM((V,), jnp.float32),
            pltpu.VMEM((HPW,), jnp.float32),
            pltpu.VMEM_SHARED((V,), jnp.float32),
            pltpu.SemaphoreType.DMA,
        ],
    )(_sc_body)
    return f(idx_ds, idx_org.reshape(NW, HPW // 128, 128),
             codebook.reshape(V // 2, 2 * C))


def _interp_body(zq_ref, par_ref, counts_ref, zhat_ref, perp_ref, w_ref):
    b = pl.program_id(0)

    @pl.when(b == 0)
    def _init():
        t2 = lax.broadcasted_iota(jnp.int32, (QD, T), 1).astype(jnp.float32)
        k2 = lax.broadcasted_iota(jnp.int32, (QD, T), 0)
        pos = jnp.clip((t2 + 0.5) * (QD / T) - 0.5, 0.0, QD - 1.0)
        lo = jnp.floor(pos)
        loi = lo.astype(jnp.int32)
        hii = jnp.minimum(loi + 1, QD - 1)
        w = pos - lo
        w_ref[...] = (jnp.where(k2 == loi, 1.0 - w, 0.0)
                      + jnp.where(k2 == hii, w, 0.0))
        cts = counts_ref[0:1, :] + counts_ref[1:2, :]
        prob = cts / jnp.sum(cts)
        ent = jnp.sum(prob * jnp.log(prob + 1e-07))
        perp_ref[...] = jnp.exp(-ent).reshape(1, 1)

    rows = zq_ref[0]
    zq = jnp.where(par_ref[0] == 0, rows[:, :C], rows[:, C:])
    zhat_ref[0] = lax.dot_general(
        zq, w_ref[...], (((0,), (0,)), ((), ())),
        precision=lax.Precision.HIGHEST,
        preferred_element_type=jnp.float32)


def _interp_call(zq, par, counts, interpret=False):
    return pl.pallas_call(
        _interp_body,
        grid=(B,),
        in_specs=[
            pl.BlockSpec((1, QD, 2 * C), lambda b: (b, 0, 0)),
            pl.BlockSpec((1, QD, 1), lambda b: (b, 0, 0)),
            pl.BlockSpec((NC, V), lambda b: (0, 0)),
        ],
        out_specs=[
            pl.BlockSpec((1, C, T), lambda b: (b, 0, 0)),
            pl.BlockSpec((1, 1), lambda b: (0, 0)),
        ],
        out_shape=[
            jax.ShapeDtypeStruct((B, C, T), jnp.float32),
            jax.ShapeDtypeStruct((1, 1), jnp.float32),
        ],
        scratch_shapes=[pltpu.VMEM((QD, T), jnp.float32)],
        interpret=interpret,
    )(zq, par, counts)


def kernel(z, codebook, q):
    cbn = _cbn_call(codebook)
    idx_org3, idx_ds3 = _nn_call(z, cbn)
    idx_org = idx_org3.reshape(-1)
    idx_ds = idx_ds3.reshape(-1) + (q - QD)
    zq, counts = _sc_call(idx_ds.astype(jnp.int32), idx_org, codebook)
    par = (idx_ds & 1).reshape(B, QD, 1)
    z_hat, perp = _interp_call(zq.reshape(B, QD, 2 * C), par, counts)
    return z_hat, perp.reshape(())
